# baseline (speedup 1.0000x reference)
import functools

import jax
import jax.numpy as jnp
from jax import lax
from jax.experimental import pallas as pl
from jax.experimental.pallas import tpu as pltpu
from jax.experimental.pallas import tpu_sc as plsc

B0, B1 = 16384, 200
NUM_E, DIM = 10, 3
TOTAL_WORDS = B0 * B1 * DIM // 2
PLANE = B0 * B1 // 2
NJB = B1 // 8
NIB = B0 // 128
IBW = NIB // 32


def _body(
    idx_hbm, wi_hbm, out_hbm, in_v, out_v, pt0_v, pt1_v, pt2_v, wi_v, sem_in, sem_out
):
    out_w = out_hbm.bitcast(jnp.int32)
    wid = lax.axis_index("s") * 2 + lax.axis_index("c")
    pts = (pt0_v, pt1_v, pt2_v)

    pltpu.sync_copy(wi_hbm, wi_v)
    lanes = lax.broadcasted_iota(jnp.int32, (16,), 0)
    for t in range(7):
        c = jnp.minimum(lanes + 16 * t, 99)
        a = c // 10
        b = c - 10 * a
        for d in range(3):
            wa = plsc.load_gather(wi_v, [3 * a + d])
            wb = plsc.load_gather(wi_v, [3 * b + d])
            plsc.store_scatter(pts[d], [c], wa | (wb << 16))

    CHUNK = IBW * 1024

    def in_desc(jb, buf):
        in_off = jb * (NIB * 1024) + wid * CHUNK
        return pltpu.make_async_copy(
            idx_hbm.at[pl.ds(in_off, CHUNK)],
            in_v.at[pl.ds(buf * CHUNK, CHUNK)],
            sem_in.at[buf],
        )

    def out_desc(jb, buf, d):
        out_row = jb * (NIB * 4) + wid * (IBW * 4)
        return pltpu.make_async_copy(
            out_v.at[pl.ds(buf * 48 + d * 16, IBW * 4), :],
            out_w.at[pl.ds(d * (PLANE // 128) + out_row, IBW * 4), :],
            sem_out.at[buf],
        )

    def compute(buf):
        ibase = buf * CHUNK
        for t in range(IBW):
            for jq2 in range(2):
                jqs = (2 * jq2, 2 * jq2 + 1)
                vas = {}
                vbs = {}
                for jq in jqs:
                    for k in range(8):
                        vas[jq, k] = in_v[
                            pl.ds(ibase + t * 1024 + (2 * jq) * 128 + 16 * k, 16)
                        ]
                        vbs[jq, k] = in_v[
                            pl.ds(
                                ibase + t * 1024 + (2 * jq + 1) * 128 + 16 * k, 16
                            )
                        ]
                cs = {
                    (jq, k): vas[jq, k] * 10 + vbs[jq, k]
                    for jq in jqs
                    for k in range(8)
                }
                for d in range(3):
                    ws = {
                        (jq, k): plsc.load_gather(pts[d], [cs[jq, k]])
                        for jq in jqs
                        for k in range(8)
                    }
                    for jq in jqs:
                        for k in range(8):
                            out_v[
                                buf * 48 + d * 16 + t * 4 + jq, pl.ds(16 * k, 16)
                            ] = ws[jq, k]

    in_desc(0, 0).start()

    def pair_step(p, _):
        jb0 = 2 * p
        jb1 = jb0 + 1
        in_desc(jb0, 0).wait()
        in_desc(jb1, 1).start()

        @pl.when(p >= 1)
        def _drain0():
            for d in range(3):
                out_desc(jb0 - 2, 0, d).wait()

        compute(0)
        for d in range(3):
            out_desc(jb0, 0, d).start()
        in_desc(jb1, 1).wait()
        in_desc(jb0 + 2, 0).start()

        @pl.when(p >= 1)
        def _drain1():
            for d in range(3):
                out_desc(jb1 - 2, 1, d).wait()

        compute(1)
        for d in range(3):
            out_desc(jb1, 1, d).start()
        return _

    lax.fori_loop(0, (NJB - 1) // 2, pair_step, None)

    in_desc(NJB - 1, 0).wait()
    for d in range(3):
        out_desc(NJB - 3, 0, d).wait()
    compute(0)
    for d in range(3):
        out_desc(NJB - 1, 0, d).start()
    for d in range(3):
        out_desc(NJB - 2, 1, d).wait()
        out_desc(NJB - 1, 0, d).wait()


_mesh = plsc.VectorSubcoreMesh(core_axis_name="c", subcore_axis_name="s")

_sc_call = functools.partial(
    pl.kernel,
    mesh=_mesh,
    out_type=jax.ShapeDtypeStruct((2 * TOTAL_WORDS // 128, 128), jnp.float16),
    scratch_types=[
        pltpu.VMEM((2 * IBW * 1024,), jnp.int32),
        pltpu.VMEM((2 * 3 * IBW * 4, 128), jnp.int32),
        pltpu.VMEM((112,), jnp.int32),
        pltpu.VMEM((112,), jnp.int32),
        pltpu.VMEM((112,), jnp.int32),
        pltpu.VMEM((32,), jnp.int32),
        pltpu.SemaphoreType.DMA((2,)),
        pltpu.SemaphoreType.DMA((2,)),
    ],
    compiler_params=pltpu.CompilerParams(needs_layout_passes=False),
)(_body)


@jax.jit
def kernel(input_tensor, weight):
    idx_flat = (
        input_tensor.reshape(128, 128, 25, 8)
        .transpose(2, 0, 3, 1)
        .reshape(-1)
    )
    w16 = jax.lax.bitcast_convert_type(weight, jnp.uint16).reshape(-1)
    wi = jnp.zeros((32,), jnp.int32).at[:30].set(w16.astype(jnp.int32))
    out_halves = _sc_call(idx_flat, wi)
    out = (
        out_halves.reshape(3, 25, 128, 4, 2, 128)
        .transpose(2, 5, 1, 3, 4, 0)
        .reshape(B0, B1, DIM)
    )
    return out


# speedup vs baseline: 1.0331x; 1.0331x over previous
import functools

import jax
import jax.numpy as jnp
from jax import lax
from jax.experimental import pallas as pl
from jax.experimental.pallas import tpu as pltpu
from jax.experimental.pallas import tpu_sc as plsc

B0, B1 = 16384, 200
NUM_E, DIM = 10, 3
TOTAL_WORDS = B0 * B1 * DIM // 2
PLANE = B0 * B1 // 2
NJB = B1 // 8
NIB = B0 // 128
IBW = NIB // 32


def _body(
    idx_hbm, wi_hbm, out_hbm, in_v, out_v, pt0_v, pt1_v, pt2_v, wi_v, sem_in, sem_out
):
    out_w = out_hbm.bitcast(jnp.int32)
    wid = lax.axis_index("s") * 2 + lax.axis_index("c")
    pts = (pt0_v, pt1_v, pt2_v)

    pltpu.sync_copy(wi_hbm, wi_v)
    lanes = lax.broadcasted_iota(jnp.int32, (16,), 0)
    for t in range(7):
        c = jnp.minimum(lanes + 16 * t, 99)
        a = c // 10
        b = c - 10 * a
        for d in range(3):
            wa = plsc.load_gather(wi_v, [3 * a + d])
            wb = plsc.load_gather(wi_v, [3 * b + d])
            v = wa | (wb << 16)
            c16 = c * 16
            for rep in range(16):
                plsc.store_scatter(pts[d], [c16 + rep], v)

    CHUNK = IBW * 1024

    def in_desc(jb, buf):
        in_off = jb * (NIB * 1024) + wid * CHUNK
        return pltpu.make_async_copy(
            idx_hbm.at[pl.ds(in_off, CHUNK)],
            in_v.at[pl.ds(buf * CHUNK, CHUNK)],
            sem_in.at[buf],
        )

    def out_desc(jb, buf, d):
        out_row = jb * (NIB * 4) + wid * (IBW * 4)
        return pltpu.make_async_copy(
            out_v.at[pl.ds(buf * 48 + d * 16, IBW * 4), :],
            out_w.at[pl.ds(d * (PLANE // 128) + out_row, IBW * 4), :],
            sem_out.at[buf],
        )

    def compute(buf):
        ibase = buf * CHUNK
        for t in range(IBW):
            for jq in range(4):
                vas = [
                    in_v[pl.ds(ibase + t * 1024 + (2 * jq) * 128 + 16 * k, 16)]
                    for k in range(8)
                ]
                vbs = [
                    in_v[
                        pl.ds(ibase + t * 1024 + (2 * jq + 1) * 128 + 16 * k, 16)
                    ]
                    for k in range(8)
                ]
                cs = [(vas[k] * 10 + vbs[k]) * 16 + lanes for k in range(8)]
                ws = [
                    [plsc.load_gather(pts[d], [cs[k]]) for d in range(3)]
                    for k in range(8)
                ]
                for d in range(3):
                    for k in range(8):
                        out_v[buf * 48 + d * 16 + t * 4 + jq, pl.ds(16 * k, 16)] = (
                            ws[k][d]
                        )

    in_desc(0, 0).start()

    def pair_step(p, _):
        jb0 = 2 * p
        jb1 = jb0 + 1
        in_desc(jb0, 0).wait()
        in_desc(jb1, 1).start()

        @pl.when(p >= 1)
        def _drain0():
            for d in range(3):
                out_desc(jb0 - 2, 0, d).wait()

        compute(0)
        for d in range(3):
            out_desc(jb0, 0, d).start()
        in_desc(jb1, 1).wait()
        in_desc(jb0 + 2, 0).start()

        @pl.when(p >= 1)
        def _drain1():
            for d in range(3):
                out_desc(jb1 - 2, 1, d).wait()

        compute(1)
        for d in range(3):
            out_desc(jb1, 1, d).start()
        return _

    lax.fori_loop(0, (NJB - 1) // 2, pair_step, None)

    in_desc(NJB - 1, 0).wait()
    for d in range(3):
        out_desc(NJB - 3, 0, d).wait()
    compute(0)
    for d in range(3):
        out_desc(NJB - 1, 0, d).start()
    for d in range(3):
        out_desc(NJB - 2, 1, d).wait()
        out_desc(NJB - 1, 0, d).wait()


_mesh = plsc.VectorSubcoreMesh(core_axis_name="c", subcore_axis_name="s")

_sc_call = functools.partial(
    pl.kernel,
    mesh=_mesh,
    out_type=jax.ShapeDtypeStruct((2 * TOTAL_WORDS // 128, 128), jnp.float16),
    scratch_types=[
        pltpu.VMEM((2 * IBW * 1024,), jnp.int32),
        pltpu.VMEM((2 * 3 * IBW * 4, 128), jnp.int32),
        pltpu.VMEM((1600,), jnp.int32),
        pltpu.VMEM((1600,), jnp.int32),
        pltpu.VMEM((1600,), jnp.int32),
        pltpu.VMEM((32,), jnp.int32),
        pltpu.SemaphoreType.DMA((2,)),
        pltpu.SemaphoreType.DMA((2,)),
    ],
    compiler_params=pltpu.CompilerParams(needs_layout_passes=False),
)(_body)


@jax.jit
def kernel(input_tensor, weight):
    idx_flat = (
        input_tensor.reshape(128, 128, 25, 8)
        .transpose(2, 0, 3, 1)
        .reshape(-1)
    )
    w16 = jax.lax.bitcast_convert_type(weight, jnp.uint16).reshape(-1)
    wi = jnp.zeros((32,), jnp.int32).at[:30].set(w16.astype(jnp.int32))
    out_halves = _sc_call(idx_flat, wi)
    out = (
        out_halves.reshape(3, 25, 128, 4, 2, 128)
        .transpose(2, 5, 1, 3, 4, 0)
        .reshape(B0, B1, DIM)
    )
    return out


# speedup vs baseline: 1.0453x; 1.0118x over previous
import functools

import jax
import jax.numpy as jnp
from jax import lax
from jax.experimental import pallas as pl
from jax.experimental.pallas import tpu as pltpu
from jax.experimental.pallas import tpu_sc as plsc

B0, B1 = 16384, 200
NUM_E, DIM = 10, 3
TOTAL_WORDS = B0 * B1 * DIM // 2
PLANE = B0 * B1 // 2
NJB = B1 // 8
NIB = B0 // 128
IBW = NIB // 32


def _body(
    idx_hbm, wi_hbm, out_hbm, in_v, out_v, pt0_v, pt1_v, pt2_v, wi_v, sem_in, sem_out
):
    out_w = out_hbm.bitcast(jnp.int32)
    wid = lax.axis_index("s") * 2 + lax.axis_index("c")
    pts = (pt0_v, pt1_v, pt2_v)

    pltpu.sync_copy(wi_hbm, wi_v)
    lanes = lax.broadcasted_iota(jnp.int32, (16,), 0)
    for t in range(7):
        c = jnp.minimum(lanes + 16 * t, 99)
        a = c // 10
        b = c - 10 * a
        for d in range(3):
            wa = plsc.load_gather(wi_v, [3 * a + d])
            wb = plsc.load_gather(wi_v, [3 * b + d])
            plsc.store_scatter(pts[d], [c], wa | (wb << 16))

    CHUNK = IBW * 1024

    def in_desc(jb, buf):
        in_off = jb * (NIB * 1024) + wid * CHUNK
        return pltpu.make_async_copy(
            idx_hbm.at[pl.ds(in_off, CHUNK)],
            in_v.at[pl.ds(buf * CHUNK, CHUNK)],
            sem_in.at[buf],
        )

    def out_desc(jb, buf, d):
        out_row = jb * (NIB * 4) + wid * (IBW * 4)
        return pltpu.make_async_copy(
            out_v.at[pl.ds(buf * 48 + d * 16, IBW * 4), :],
            out_w.at[pl.ds(d * (PLANE // 128) + out_row, IBW * 4), :],
            sem_out.at[buf],
        )

    def compute(buf):
        ibase = buf * CHUNK
        for t in range(IBW):
            for jq in range(4):
                vas = [
                    in_v[pl.ds(ibase + t * 1024 + (2 * jq) * 128 + 16 * k, 16)]
                    for k in range(8)
                ]
                vbs = [
                    in_v[
                        pl.ds(ibase + t * 1024 + (2 * jq + 1) * 128 + 16 * k, 16)
                    ]
                    for k in range(8)
                ]
                cs = [vas[k] * 10 + vbs[k] for k in range(8)]
                ws = [
                    [plsc.load_gather(pts[d], [cs[k]]) for d in range(3)]
                    for k in range(8)
                ]
                for d in range(3):
                    for k in range(8):
                        out_v[buf * 48 + d * 16 + t * 4 + jq, pl.ds(16 * k, 16)] = (
                            ws[k][d]
                        )

    in_desc(0, 0).start()

    def pair_step(p, _):
        jb0 = 2 * p
        jb1 = jb0 + 1
        in_desc(jb0, 0).wait()
        in_desc(jb1, 1).start()

        @pl.when(p >= 1)
        def _drain0():
            for d in range(3):
                out_desc(jb0 - 2, 0, d).wait()

        compute(0)
        for d in range(3):
            out_desc(jb0, 0, d).start()
        in_desc(jb1, 1).wait()
        in_desc(jb0 + 2, 0).start()

        @pl.when(p >= 1)
        def _drain1():
            for d in range(3):
                out_desc(jb1 - 2, 1, d).wait()

        compute(1)
        for d in range(3):
            out_desc(jb1, 1, d).start()
        return _

    lax.fori_loop(0, (NJB - 1) // 2, pair_step, None)

    in_desc(NJB - 1, 0).wait()
    for d in range(3):
        out_desc(NJB - 3, 0, d).wait()
    compute(0)
    for d in range(3):
        out_desc(NJB - 1, 0, d).start()
    for d in range(3):
        out_desc(NJB - 2, 1, d).wait()
        out_desc(NJB - 1, 0, d).wait()


_mesh = plsc.VectorSubcoreMesh(core_axis_name="c", subcore_axis_name="s")

_sc_call = functools.partial(
    pl.kernel,
    mesh=_mesh,
    out_type=jax.ShapeDtypeStruct((2 * TOTAL_WORDS // 128, 128), jnp.float16),
    scratch_types=[
        pltpu.VMEM((2 * IBW * 1024,), jnp.int32),
        pltpu.VMEM((2 * 3 * IBW * 4, 128), jnp.int32),
        pltpu.VMEM((112,), jnp.int32),
        pltpu.VMEM((112,), jnp.int32),
        pltpu.VMEM((112,), jnp.int32),
        pltpu.VMEM((32,), jnp.int32),
        pltpu.SemaphoreType.DMA((2,)),
        pltpu.SemaphoreType.DMA((2,)),
    ],
    compiler_params=pltpu.CompilerParams(needs_layout_passes=False),
)(_body)


@jax.jit
def kernel(input_tensor, weight):
    idx_flat = (
        input_tensor.reshape(128, 128, 25, 8)
        .transpose(2, 0, 3, 1)
        .reshape(-1)
    )
    w16 = jax.lax.bitcast_convert_type(weight, jnp.uint16).reshape(-1)
    wi = jnp.zeros((32,), jnp.int32).at[:30].set(w16.astype(jnp.int32))
    out_halves = _sc_call(idx_flat, wi)
    out = (
        out_halves.reshape(3, 25, 128, 4, 2, 128)
        .transpose(2, 5, 1, 3, 4, 0)
        .reshape(B0, B1, DIM)
    )
    return out


# speedup vs baseline: 1.0590x; 1.0132x over previous
import functools

import jax
import jax.numpy as jnp
from jax import lax
from jax.experimental import pallas as pl
from jax.experimental.pallas import tpu as pltpu
from jax.experimental.pallas import tpu_sc as plsc

B0, B1 = 16384, 200
NUM_E, DIM = 10, 3
TOTAL_WORDS = B0 * B1 * DIM // 2
PLANE = B0 * B1 // 2
NJB = B1 // 8
NIB = B0 // 128
IBW = NIB // 32


def _body(
    idx_hbm, wi_hbm, out_hbm, in_v, out_v, pt0_v, pt1_v, pt2_v, wi_v, sem_in, sem_out
):
    out_w = out_hbm.bitcast(jnp.int32)
    wid = lax.axis_index("s") * 2 + lax.axis_index("c")
    pts = (pt0_v, pt1_v, pt2_v)

    CHUNK = IBW * 1024

    def in_desc(jb, buf):
        in_off = jb * (NIB * 1024) + wid * CHUNK
        return pltpu.make_async_copy(
            idx_hbm.at[pl.ds(in_off, CHUNK)],
            in_v.at[pl.ds(buf * CHUNK, CHUNK)],
            sem_in.at[buf],
        )

    in_desc(0, 0).start()

    pltpu.sync_copy(wi_hbm, wi_v)
    lanes = lax.broadcasted_iota(jnp.int32, (16,), 0)
    for t in range(7):
        c = jnp.minimum(lanes + 16 * t, 99)
        a = c // 10
        b = c - 10 * a
        for d in range(3):
            wa = plsc.load_gather(wi_v, [3 * a + d])
            wb = plsc.load_gather(wi_v, [3 * b + d])
            plsc.store_scatter(pts[d], [c], wa | (wb << 16))

    def out_desc(jb, buf, d):
        out_row = jb * (NIB * 4) + wid * (IBW * 4)
        return pltpu.make_async_copy(
            out_v.at[pl.ds(buf * 48 + d * 16, IBW * 4), :],
            out_w.at[pl.ds(d * (PLANE // 128) + out_row, IBW * 4), :],
            sem_out.at[buf],
        )

    def compute(buf):
        ibase = buf * CHUNK
        for t in range(IBW):
            for jq in range(4):
                vas = [
                    in_v[pl.ds(ibase + t * 1024 + (2 * jq) * 128 + 16 * k, 16)]
                    for k in range(8)
                ]
                vbs = [
                    in_v[
                        pl.ds(ibase + t * 1024 + (2 * jq + 1) * 128 + 16 * k, 16)
                    ]
                    for k in range(8)
                ]
                cs = [vas[k] * 10 + vbs[k] for k in range(8)]
                ws = [
                    [plsc.load_gather(pts[d], [cs[k]]) for d in range(3)]
                    for k in range(8)
                ]
                for d in range(3):
                    for k in range(8):
                        out_v[buf * 48 + d * 16 + t * 4 + jq, pl.ds(16 * k, 16)] = (
                            ws[k][d]
                        )

    def pair_step(p, _):
        jb0 = 2 * p
        jb1 = jb0 + 1
        in_desc(jb0, 0).wait()
        in_desc(jb1, 1).start()

        @pl.when(p >= 1)
        def _drain0():
            for d in range(3):
                out_desc(jb0 - 2, 0, d).wait()

        compute(0)
        for d in range(3):
            out_desc(jb0, 0, d).start()
        in_desc(jb1, 1).wait()
        in_desc(jb0 + 2, 0).start()

        @pl.when(p >= 1)
        def _drain1():
            for d in range(3):
                out_desc(jb1 - 2, 1, d).wait()

        compute(1)
        for d in range(3):
            out_desc(jb1, 1, d).start()
        return _

    lax.fori_loop(0, (NJB - 1) // 2, pair_step, None)

    in_desc(NJB - 1, 0).wait()
    for d in range(3):
        out_desc(NJB - 3, 0, d).wait()
    compute(0)
    for d in range(3):
        out_desc(NJB - 1, 0, d).start()
    for d in range(3):
        out_desc(NJB - 2, 1, d).wait()
        out_desc(NJB - 1, 0, d).wait()


_mesh = plsc.VectorSubcoreMesh(core_axis_name="c", subcore_axis_name="s")

_sc_call = functools.partial(
    pl.kernel,
    mesh=_mesh,
    out_type=jax.ShapeDtypeStruct((2 * TOTAL_WORDS // 128, 128), jnp.float16),
    scratch_types=[
        pltpu.VMEM((2 * IBW * 1024,), jnp.int32),
        pltpu.VMEM((2 * 3 * IBW * 4, 128), jnp.int32),
        pltpu.VMEM((112,), jnp.int32),
        pltpu.VMEM((112,), jnp.int32),
        pltpu.VMEM((112,), jnp.int32),
        pltpu.VMEM((32,), jnp.int32),
        pltpu.SemaphoreType.DMA((2,)),
        pltpu.SemaphoreType.DMA((2,)),
    ],
    compiler_params=pltpu.CompilerParams(needs_layout_passes=False),
)(_body)


@jax.jit
def kernel(input_tensor, weight):
    idx_flat = (
        input_tensor.reshape(128, 128, 25, 8)
        .transpose(2, 0, 3, 1)
        .reshape(-1)
    )
    w16 = jax.lax.bitcast_convert_type(weight, jnp.uint16).reshape(-1)
    wi = jnp.zeros((32,), jnp.int32).at[:30].set(w16.astype(jnp.int32))
    out_halves = _sc_call(idx_flat, wi)
    out = (
        out_halves.reshape(3, 25, 128, 4, 2, 128)
        .transpose(2, 5, 1, 3, 4, 0)
        .reshape(B0, B1, DIM)
    )
    return out
